# baseline (speedup 1.0000x reference)
import functools

import jax
import jax.numpy as jnp
from jax.experimental import pallas as pl
from jax.experimental.pallas import tpu as pltpu

B = 512
OUT_DIM = 128
N_PROT = 2000


def _ceil_to(x, m):
    return (x + m - 1) // m * m


@functools.partial(jax.jit, static_argnames=("act", "m_blk"))
def _mm_bias(x, w, b, act="none", m_blk=512):
    M, K = x.shape
    N = w.shape[1]
    Mp = _ceil_to(M, m_blk)
    if Mp != M:
        x = jnp.pad(x, ((0, Mp - M), (0, 0)))

    def body(x_ref, w_ref, b_ref, o_ref):
        acc = jnp.dot(x_ref[...], w_ref[...], preferred_element_type=jnp.float32)
        acc = acc + b_ref[...][None, :]
        if act == "relu":
            acc = jnp.maximum(acc, 0.0)
        o_ref[...] = acc

    out = pl.pallas_call(
        body,
        grid=(Mp // m_blk,),
        in_specs=[
            pl.BlockSpec((m_blk, K), lambda i: (i, 0)),
            pl.BlockSpec((K, N), lambda i: (0, 0)),
            pl.BlockSpec((N,), lambda i: (0,)),
        ],
        out_specs=pl.BlockSpec((m_blk, N), lambda i: (i, 0)),
        out_shape=jax.ShapeDtypeStruct((Mp, N), jnp.float32),
    )(x, w, b)
    return out[:M]


def _gcn(x, ei, W, b):
    n = x.shape[0]
    loop = jnp.arange(n, dtype=ei.dtype)
    src = jnp.concatenate([ei[0], loop])
    dst = jnp.concatenate([ei[1], loop])
    deg = jnp.zeros((n,), x.dtype).at[dst].add(1.0)
    dinv = jnp.where(deg > 0, 1.0 / jnp.sqrt(deg), 0.0)
    norm = (dinv[src] * dinv[dst])[:, None]
    h = _mm_bias(x, W, jnp.zeros((W.shape[1],), x.dtype))
    out = jnp.zeros((n, W.shape[1]), x.dtype).at[dst].add(h[src] * norm)
    return out + b


def _gep(x, batch, ns):
    s = jax.ops.segment_sum(x, batch, num_segments=ns)
    c = jax.ops.segment_sum(jnp.ones((x.shape[0],), x.dtype), batch, num_segments=ns)
    return s / jnp.maximum(c, 1.0)[:, None]


def _bn_eval(x, g, b):
    return x / jnp.sqrt(1.0 + 1e-5) * g + b


def kernel(mol_x, mol_edge_index, mol_batch, seq_num, ppi_edge, ppi_features, p_x, p_edge_index, p_edge_len, p_batch, params):
    relu = jax.nn.relu
    P = params
    x = relu(_gcn(mol_x, mol_edge_index, P["molW1"], P["molb1"]))
    x = relu(_gcn(x, mol_edge_index, P["molW2"], P["molb2"]))
    x = relu(_gcn(x, mol_edge_index, P["molW3"], P["molb3"]))
    x = _gep(x, mol_batch, B)
    x = _mm_bias(x, P["molFC1W"], P["molFC1b"], act="relu")
    x = _mm_bias(x, P["molFC2W"], P["molFC2b"])

    p = _bn_eval(relu(_gcn(p_x, p_edge_index, P["proW1"], P["prob1"])), P["bn1_g"], P["bn1_b"])
    p = _bn_eval(relu(_gcn(p, p_edge_index, P["proW2"], P["prob2"])), P["bn2_g"], P["bn2_b"])
    p = _bn_eval(relu(_gcn(p, p_edge_index, P["proW3"], P["prob3"])), P["bn3_g"], P["bn3_b"])
    p = _gep(p, p_batch, N_PROT)
    p = _mm_bias(p, P["proFC1W"], P["proFC1b"], act="relu")
    p = _mm_bias(p, P["proFC2W"], P["proFC2b"])

    ppi = relu(_gcn(p, ppi_edge, P["ppiW1"], P["ppib1"]))
    ppi = relu(_gcn(ppi, ppi_edge, P["ppiW2"], P["ppib2"]))
    ppi = _mm_bias(ppi, P["ppiFC1W"], P["ppiFC1b"], act="relu")
    ppi = _mm_bias(ppi, P["ppiFC2W"], P["ppiFC2b"])
    ppi = ppi[seq_num]

    xc = jnp.concatenate([x, ppi], axis=1)
    xc = _mm_bias(xc, P["fc1W"], P["fc1b"], act="relu")
    xc = _mm_bias(xc, P["fc2W"], P["fc2b"], act="relu")
    return _mm_bias(xc, P["outW"], P["outb"])


# speedup vs baseline: 7.0997x; 7.0997x over previous
import functools

import jax
import jax.numpy as jnp
from jax import lax
from jax.experimental import pallas as pl
from jax.experimental.pallas import tpu as pltpu
from jax.experimental.pallas import tpu_sc as plsc

B = 512
OUT_DIM = 128
N_PROT = 2000

_NC, _NS, _SUB = 2, 16, 128


def _ceil_to(x, m):
    return (x + m - 1) // m * m


@functools.partial(jax.jit, static_argnames=("act", "m_blk"))
def _mm_bias(x, w, b, act="none", m_blk=512):
    M, K = x.shape
    N = w.shape[1]
    Mp = _ceil_to(M, m_blk)
    if Mp != M:
        x = jnp.pad(x, ((0, Mp - M), (0, 0)))

    def body(x_ref, w_ref, b_ref, o_ref):
        acc = jnp.dot(x_ref[...], w_ref[...], preferred_element_type=jnp.float32)
        acc = acc + b_ref[...][None, :]
        if act == "relu":
            acc = jnp.maximum(acc, 0.0)
        o_ref[...] = acc

    out = pl.pallas_call(
        body,
        grid=(Mp // m_blk,),
        in_specs=[
            pl.BlockSpec((m_blk, K), lambda i: (i, 0)),
            pl.BlockSpec((K, N), lambda i: (0, 0)),
            pl.BlockSpec((N,), lambda i: (0,)),
        ],
        out_specs=pl.BlockSpec((m_blk, N), lambda i: (i, 0)),
        out_shape=jax.ShapeDtypeStruct((Mp, N), jnp.float32),
    )(x, w, b)
    return out[:M]


def _sc_mesh():
    return plsc.VectorSubcoreMesh(
        core_axis_name="c", subcore_axis_name="s",
        num_cores=_NC, num_subcores=_NS)


def _fill_zero_rows(ref, nrows, ncols):
    z = jnp.zeros((16,), jnp.float32)

    def body(i, carry):
        for j in range(ncols // 16):
            ref[i, pl.ds(j * 16, 16)] = z
        return carry

    lax.fori_loop(0, nrows, body, 0)


def _sc_propagate(g, src2d, dst2d, np_, dc, nsub, zr):
    total_rows = src2d.shape[0]
    rows_half = total_rows // _NC
    rows_sub = rows_half // _NS
    n_chunks = rows_sub // nsub
    assert rows_sub % nsub == 0 and np_ % _NS == 0
    rps = np_ // _NS
    assert rps % zr == 0

    @functools.partial(
        pl.kernel,
        out_type=jax.ShapeDtypeStruct((_NC * np_, dc), jnp.float32),
        mesh=_sc_mesh(),
        compiler_params=pltpu.CompilerParams(use_tc_tiling_on_sc=False),
        scratch_types=[
            pltpu.VMEM((nsub, _SUB), jnp.int32),
            pltpu.VMEM((nsub, _SUB), jnp.int32),
            pltpu.VMEM((nsub, _SUB, dc), jnp.float32),
            pltpu.VMEM((zr, dc), jnp.float32),
            pltpu.VMEM_SHARED((np_, dc), jnp.float32),
            pltpu.SemaphoreType.DMA,
        ],
    )
    def k(g_r, src_r, dst_r, out_r, sidx, didx, rows, zbuf, table, sem):
        c = lax.axis_index("c")
        s = lax.axis_index("s")
        _fill_zero_rows(zbuf, zr, dc)
        r0 = s * rps
        rb0 = c * rows_half + s * rows_sub
        for zi in range(rps // zr):
            pltpu.sync_copy(zbuf, table.at[pl.ds(r0 + zi * zr, zr)])
        plsc.subcore_barrier()

        def chunk(i, carry):
            rb = rb0 + i * nsub
            pltpu.sync_copy(src_r.at[pl.ds(rb, nsub)], sidx)
            pltpu.sync_copy(dst_r.at[pl.ds(rb, nsub)], didx)
            cps = [pltpu.async_copy(g_r.at[sidx.at[j]], rows.at[j], sem)
                   for j in range(nsub)]
            for cp in cps:
                cp.wait()
            for j in range(nsub):
                pltpu.sync_copy(rows.at[j], table.at[didx.at[j]], add=True)
            return carry

        lax.fori_loop(0, n_chunks, chunk, 0)
        plsc.subcore_barrier()
        pltpu.sync_copy(table.at[pl.ds(r0, rps)],
                        out_r.at[pl.ds(c * np_ + r0, rps)])

    return k(g, src2d, dst2d)


def _sc_hist(idx2d, np_):
    total_rows = idx2d.shape[0]
    rows_half = total_rows // _NC
    rows_sub = rows_half // _NS
    nsub = 8 if rows_sub % 8 == 0 else 1
    n_chunks = rows_sub // nsub
    rps = np_ // _NS
    assert rps % 16 == 0

    @functools.partial(
        pl.kernel,
        out_type=jax.ShapeDtypeStruct((_NC * np_,), jnp.float32),
        mesh=_sc_mesh(),
        compiler_params=pltpu.CompilerParams(use_tc_tiling_on_sc=False),
        scratch_types=[
            pltpu.VMEM((nsub, _SUB), jnp.int32),
            pltpu.VMEM((_SUB,), jnp.float32),
            pltpu.VMEM((rps,), jnp.float32),
            pltpu.VMEM_SHARED((np_,), jnp.float32),
        ],
    )
    def k(idx_r, out_r, didx, ones, zrow, table):
        c = lax.axis_index("c")
        s = lax.axis_index("s")
        one = jnp.ones((16,), jnp.float32)
        zero = jnp.zeros((16,), jnp.float32)
        for j in range(_SUB // 16):
            ones[pl.ds(j * 16, 16)] = one

        def zb(i, carry):
            zrow[pl.ds(i * 16, 16)] = zero
            return carry

        lax.fori_loop(0, rps // 16, zb, 0)
        r0 = s * rps
        pltpu.sync_copy(zrow, table.at[pl.ds(r0, rps)])
        plsc.subcore_barrier()
        rb0 = c * rows_half + s * rows_sub

        def chunk(i, carry):
            pltpu.sync_copy(idx_r.at[pl.ds(rb0 + i * nsub, nsub)], didx)
            for j in range(nsub):
                pltpu.sync_copy(ones, table.at[didx.at[j]], add=True)
            return carry

        lax.fori_loop(0, n_chunks, chunk, 0)
        plsc.subcore_barrier()
        pltpu.sync_copy(table.at[pl.ds(r0, rps)], zrow)
        pltpu.sync_copy(zrow, out_r.at[pl.ds(c * np_ + r0, rps)])

    return k(idx2d)


def _sc_pool(xblk, batch2d, np_):
    d = xblk.shape[1]
    total_rows = batch2d.shape[0]
    rows_half = total_rows // _NC
    rows_sub = rows_half // _NS
    rps = np_ // _NS

    @functools.partial(
        pl.kernel,
        out_type=jax.ShapeDtypeStruct((_NC * np_, d), jnp.float32),
        mesh=_sc_mesh(),
        compiler_params=pltpu.CompilerParams(use_tc_tiling_on_sc=False),
        scratch_types=[
            pltpu.VMEM((1, _SUB), jnp.int32),
            pltpu.VMEM((_SUB, 32), jnp.float32),
            pltpu.VMEM((128, 32), jnp.float32),
            pltpu.VMEM_SHARED((np_, 32), jnp.float32),
        ],
    )
    def k(x_r, b_r, out_r, didx, rows, zbuf, table):
        c = lax.axis_index("c")
        s = lax.axis_index("s")
        _fill_zero_rows(zbuf, 128, 32)
        r0 = s * rps
        for zi in range(rps // 128):
            pltpu.sync_copy(zbuf, table.at[pl.ds(r0 + zi * 128, 128)])
        plsc.subcore_barrier()
        rb0 = c * rows_half + s * rows_sub

        def chunk(i, carry):
            rb = rb0 + i
            pltpu.sync_copy(b_r.at[pl.ds(rb, 1)], didx)
            pltpu.sync_copy(x_r.at[pl.ds(rb * _SUB, _SUB)], rows)
            pltpu.sync_copy(rows, table.at[didx.at[0]], add=True)
            return carry

        lax.fori_loop(0, rows_sub, chunk, 0)
        plsc.subcore_barrier()
        pltpu.sync_copy(table.at[pl.ds(r0, rps)],
                        out_r.at[pl.ds(c * np_ + r0, rps)])

    return k(xblk, batch2d)


def _pad_edges(ei, n, ep):
    e = ei.shape[1]
    pad = ep - e
    ar = jnp.arange(pad, dtype=jnp.int32)
    src = jnp.concatenate([ei[0].astype(jnp.int32), ar % 256])
    dst = jnp.concatenate([ei[1].astype(jnp.int32), n + (ar % 16)])
    return src.reshape(-1, _SUB), dst.reshape(-1, _SUB)


def _blocks(x, dinv_p, np_, dpad, bw=16):
    n, d = x.shape
    g = jnp.pad(x, ((0, np_ - n), (0, dpad - d))) * dinv_p[:, None]
    return [g[:, i * bw:(i + 1) * bw] for i in range(dpad // bw)], g


def _combine(parts, g, dinv_p):
    np_ = g.shape[0]
    s = jnp.concatenate([p[:np_] + p[np_:] for p in parts], axis=1)
    return dinv_p[:, None] * (s + g)


def _degree_dinv(dst2d, n, np_):
    parts = _sc_hist(dst2d, np_)
    deg = parts[:n] + parts[np_:np_ + n] + 1.0
    dinv = lax.rsqrt(deg)
    return jnp.pad(dinv, (0, np_ - n))


def _gcn_prop(g_blocks, src2d, dst2d, np_, nsub=8, zr=None):
    if zr is None:
        zr = np_ // _NS
        while zr > 512:
            zr //= 2
    return [_sc_propagate(g, src2d, dst2d, np_, g.shape[1], nsub, zr)
            for g in g_blocks]


def _sorted_counts(batch, ns):
    idx = jnp.arange(ns + 1, dtype=batch.dtype)
    bounds = jnp.searchsorted(batch, idx)
    return (bounds[1:] - bounds[:-1]).astype(jnp.float32)


def _pool_mean(x_np, batch, n, ns):
    ns_pad = 2048
    np_rows = _ceil_to(x_np.shape[0], _NC * _NS * _SUB)
    xp = jnp.pad(x_np, ((0, np_rows - x_np.shape[0]), (0, 0)))
    ar = jnp.arange(np_rows - n, dtype=jnp.int32)
    bp = jnp.concatenate([batch.astype(jnp.int32), ns + (ar % 16)])
    b2d = bp.reshape(-1, _SUB)
    cols = []
    for i in range(x_np.shape[1] // 32):
        parts = _sc_pool(xp[:, i * 32:(i + 1) * 32], b2d, ns_pad)
        cols.append(parts[:ns] + parts[ns_pad:ns_pad + ns])
    ssum = jnp.concatenate(cols, axis=1)
    cnt = _sorted_counts(batch, ns)
    return ssum / jnp.maximum(cnt, 1.0)[:, None]


def _bn_eval(x, g, b):
    return x / jnp.sqrt(1.0 + 1e-5) * g + b


def _pw(w, rows, cols=None):
    cols = cols if cols is not None else w.shape[1]
    return jnp.pad(w, ((0, rows - w.shape[0]), (0, cols - w.shape[1])))


def _pb(b, cols):
    return jnp.pad(b, (0, cols - b.shape[0]))


def kernel(mol_x, mol_edge_index, mol_batch, seq_num, ppi_edge, ppi_features,
           p_x, p_edge_index, p_edge_len, p_batch, params):
    relu = jax.nn.relu
    P = params

    N = mol_x.shape[0]
    NP = _ceil_to(N + 16, 448)
    EP = _ceil_to(mol_edge_index.shape[1], _NC * _NS * _SUB * 8)

    msrc, mdst = _pad_edges(mol_edge_index, N, EP)
    dinv_m = _degree_dinv(mdst, N, NP)

    gb, g = _blocks(mol_x, dinv_m, NP, 96)
    z = _combine(_gcn_prop(gb, msrc, mdst, NP), g, dinv_m)
    x1 = _mm_bias(z, _pw(P["molW1"], 96, 160), _pb(P["molb1"], 160), act="relu")

    gb, g = _blocks(x1, dinv_m, NP, 160)
    z = _combine(_gcn_prop(gb, msrc, mdst, NP), g, dinv_m)
    x2 = _mm_bias(z, _pw(P["molW2"], 160, 320), _pb(P["molb2"], 320), act="relu")

    h3 = _mm_bias(x2, _pw(P["molW3"], 320), jnp.zeros((OUT_DIM,), jnp.float32))
    gb, g = _blocks(h3, dinv_m, NP, 128)
    x3 = relu(_combine(_gcn_prop(gb, msrc, mdst, NP), g, dinv_m) + P["molb3"])
    xm = _pool_mean(x3, mol_batch, N, B)
    xm = _mm_bias(xm, P["molFC1W"], P["molFC1b"], act="relu")
    xm = _mm_bias(xm, P["molFC2W"], P["molFC2b"])

    psrc, pdst = _pad_edges(p_edge_index, N, EP)
    dinv_p = _degree_dinv(pdst, N, NP)

    gb, g = _blocks(p_x, dinv_p, NP, 64)
    z = _combine(_gcn_prop(gb, psrc, pdst, NP), g, dinv_p)
    p1 = _bn_eval(_mm_bias(z, _pw(P["proW1"], 64), P["prob1"], act="relu"),
                  P["bn1_g"], P["bn1_b"])

    h = _mm_bias(p1, P["proW2"], jnp.zeros((OUT_DIM,), jnp.float32))
    gb, g = _blocks(h, dinv_p, NP, 128)
    p2 = _bn_eval(relu(_combine(_gcn_prop(gb, psrc, pdst, NP), g, dinv_p)
                       + P["prob2"]), P["bn2_g"], P["bn2_b"])

    h = _mm_bias(p2, P["proW3"], jnp.zeros((OUT_DIM,), jnp.float32))
    gb, g = _blocks(h, dinv_p, NP, 128)
    p3 = _bn_eval(relu(_combine(_gcn_prop(gb, psrc, pdst, NP), g, dinv_p)
                       + P["prob3"]), P["bn3_g"], P["bn3_b"])
    pm = _pool_mean(p3, p_batch, N, N_PROT)
    pm = _mm_bias(pm, P["proFC1W"], P["proFC1b"], act="relu")
    pm = _mm_bias(pm, P["proFC2W"], P["proFC2b"])

    NPI = 2048
    EPI = _ceil_to(ppi_edge.shape[1], _NC * _NS * _SUB * 2)
    isrc, idst = _pad_edges(ppi_edge, N_PROT, EPI)
    dinv_i = _degree_dinv(idst, N_PROT, NPI)

    gb, g = _blocks(pm, dinv_i, NPI, 128, bw=32)
    z = _combine(_gcn_prop(gb, isrc, idst, NPI, nsub=6, zr=128), g, dinv_i)
    q1 = _mm_bias(z, P["ppiW1"], P["ppib1"], act="relu")

    h = _mm_bias(q1, P["ppiW2"], jnp.zeros((OUT_DIM,), jnp.float32))
    gb, g = _blocks(h, dinv_i, NPI, 128, bw=32)
    q2 = relu(_combine(_gcn_prop(gb, isrc, idst, NPI, nsub=6, zr=128), g, dinv_i)
              + P["ppib2"])
    q = _mm_bias(q2, P["ppiFC1W"], P["ppiFC1b"], act="relu")
    q = _mm_bias(q, P["ppiFC2W"], P["ppiFC2b"])
    q = q[seq_num]

    xc = jnp.concatenate([xm, q], axis=1)
    xc = _mm_bias(xc, P["fc1W"], P["fc1b"], act="relu")
    xc = _mm_bias(xc, P["fc2W"], P["fc2b"], act="relu")
    return _mm_bias(xc, P["outW"], P["outb"])
